# Initial kernel scaffold; baseline (speedup 1.0000x reference)
import jax
import jax.numpy as jnp
from jax.experimental import pallas as pl


def kernel(x, edge_index, W1, b1, W2, b2):
    raise NotImplementedError("write your pallas kernel here")


# speedup vs baseline: 9.5636x; 9.5636x over previous
import functools

import jax
import jax.numpy as jnp
from jax import lax
from jax.experimental import pallas as pl
from jax.experimental.pallas import tpu as pltpu
import jax.experimental.pallas.tpu_sc as plsc

N = 10000
E = 160000
D = 256
HALF = 128
CH = 128
NCHUNK = E // CH
NCORE = 2
NSUB = 16
BLK = 80
NBLK = N // BLK
ROWBLK = 1000

_mesh = plsc.VectorSubcoreMesh(core_axis_name="c", subcore_axis_name="s")


def _zero_fill(ref, rows, cols):
    per_row = cols // 16

    def body(k, _):
        ref[k // per_row, pl.ds((k % per_row) * 16, 16)] = jnp.zeros(
            (16,), jnp.float32)
        return 0

    lax.fori_loop(0, rows * per_row, body, 0)


def _nsplit(total, s):
    return jnp.where(s < (total % NSUB), total // NSUB + 1, total // NSUB)


@functools.partial(
    pl.kernel,
    out_type=jax.ShapeDtypeStruct((NCORE * N, HALF), jnp.float32),
    mesh=_mesh,
    scratch_types=[
        pltpu.VMEM((CH,), jnp.int32),
        pltpu.VMEM((CH, HALF), jnp.float32),
        pltpu.VMEM((BLK, HALF), jnp.float32),
        pltpu.VMEM_SHARED((N, HALF), jnp.float32),
    ],
)
def _deg_kernel(dst_hbm, out_hbm, dst_v, ones_v, z_v, acc_sh):
    c = lax.axis_index("c")
    s = lax.axis_index("s")
    _zero_fill(z_v, BLK, HALF)

    def ones_body(k, _):
        ones_v[k // 8, pl.ds((k % 8) * 16, 16)] = jnp.ones((16,), jnp.float32)
        return 0

    lax.fori_loop(0, CH * 8, ones_body, 0)

    def zero_acc(k, _):
        off = pl.multiple_of((k * NSUB + s) * BLK, 8)
        pltpu.sync_copy(z_v, acc_sh.at[pl.ds(off, BLK)])
        return 0

    lax.fori_loop(0, _nsplit(NBLK, s), zero_acc, 0)
    plsc.subcore_barrier()

    half = NCHUNK // NCORE

    def body(k, _):
        off = pl.multiple_of((c * half + k * NSUB + s) * CH, 8)
        pltpu.sync_copy(dst_hbm.at[pl.ds(off, CH)], dst_v)
        pltpu.sync_copy(ones_v, acc_sh.at[dst_v], add=True)
        return 0

    lax.fori_loop(0, _nsplit(half, s), body, 0)
    plsc.subcore_barrier()

    def wout(k, _):
        off = pl.multiple_of((k * NSUB + s) * BLK, 8)
        dst_off = pl.multiple_of(c * N + (k * NSUB + s) * BLK, 8)
        pltpu.sync_copy(acc_sh.at[pl.ds(off, BLK)],
                        out_hbm.at[pl.ds(dst_off, BLK)])
        return 0

    lax.fori_loop(0, _nsplit(NBLK, s), wout, 0)


@functools.partial(
    pl.kernel,
    out_type=jax.ShapeDtypeStruct((NCORE * N, HALF), jnp.float32),
    mesh=_mesh,
    scratch_types=[
        pltpu.VMEM((CH,), jnp.int32),
        pltpu.VMEM((CH,), jnp.int32),
        pltpu.VMEM((CH, HALF), jnp.float32),
        pltpu.VMEM((BLK, HALF), jnp.float32),
        pltpu.VMEM_SHARED((N, HALF), jnp.float32),
        pltpu.SemaphoreType.DMA,
    ],
)
def _agg_kernel(tab_hbm, src_hbm, dst_hbm, out_hbm,
                src_v, dst_v, rows_v, z_v, acc_sh, sem):
    c = lax.axis_index("c")
    s = lax.axis_index("s")
    _zero_fill(z_v, BLK, HALF)

    def zero_acc(k, _):
        off = pl.multiple_of((k * NSUB + s) * BLK, 8)
        pltpu.sync_copy(z_v, acc_sh.at[pl.ds(off, BLK)])
        return 0

    lax.fori_loop(0, _nsplit(NBLK, s), zero_acc, 0)
    plsc.subcore_barrier()

    base = c * N

    def body(k, _):
        off = pl.multiple_of((k * NSUB + s) * CH, 8)
        pltpu.sync_copy(src_hbm.at[pl.ds(off, CH)], src_v)
        pltpu.sync_copy(dst_hbm.at[pl.ds(off, CH)], dst_v)

        def shift(j, _):
            src_v[pl.ds(j * 16, 16)] = src_v[pl.ds(j * 16, 16)] + base
            return 0

        lax.fori_loop(0, CH // 16, shift, 0)
        pltpu.async_copy(tab_hbm.at[src_v], rows_v, sem).wait()
        pltpu.sync_copy(rows_v, acc_sh.at[dst_v], add=True)
        return 0

    lax.fori_loop(0, _nsplit(NCHUNK, s), body, 0)
    plsc.subcore_barrier()

    def wout(k, _):
        off = pl.multiple_of((k * NSUB + s) * BLK, 8)
        dst_off = pl.multiple_of(c * N + (k * NSUB + s) * BLK, 8)
        pltpu.sync_copy(acc_sh.at[pl.ds(off, BLK)],
                        out_hbm.at[pl.ds(dst_off, BLK)])
        return 0

    lax.fori_loop(0, _nsplit(NBLK, s), wout, 0)


def _tca_body(x_ref, w_ref, degp_ref, hhat_ref, self_ref, dinv_ref):
    h = jnp.dot(x_ref[...], w_ref[...], preferred_element_type=jnp.float32)
    degp = degp_ref[...]
    deg = degp[0, :, 0] + degp[1, :, 0] + 1.0
    dinv = lax.rsqrt(deg)[:, None]
    hh = h * dinv
    hhat_ref[0] = hh[:, :HALF]
    hhat_ref[1] = hh[:, HALF:]
    self_ref[...] = hh * dinv
    dinv_ref[...] = jnp.broadcast_to(dinv, (ROWBLK, HALF))


def _tcb_body(agg_ref, self_ref, dinv_ref, w_ref, b_ref, hhat_ref, self2_ref):
    agg = jnp.concatenate([agg_ref[0], agg_ref[1]], axis=1)
    dinv = dinv_ref[:, :1]
    u = jnp.maximum(agg * dinv + self_ref[...] + b_ref[...], 0.0)
    h2 = jnp.dot(u, w_ref[...], preferred_element_type=jnp.float32)
    hh2 = h2 * dinv
    hhat_ref[0] = hh2[:, :HALF]
    hhat_ref[1] = hh2[:, HALF:]
    self2_ref[...] = hh2 * dinv


def _tcc_body(agg_ref, self_ref, dinv_ref, b_ref, out_ref):
    agg = jnp.concatenate([agg_ref[0], agg_ref[1]], axis=1)
    dinv = dinv_ref[:, :1]
    out_ref[...] = agg * dinv + self_ref[...] + b_ref[...]


_GRID = N // ROWBLK

_row_spec = pl.BlockSpec((ROWBLK, D), lambda i: (i, 0))
_half2_spec = pl.BlockSpec((2, ROWBLK, HALF), lambda i: (0, i, 0))
_dinv_spec = pl.BlockSpec((ROWBLK, HALF), lambda i: (i, 0))
_w_spec = pl.BlockSpec((D, D), lambda i: (0, 0))
_b_spec = pl.BlockSpec((1, D), lambda i: (0, 0))
_degp_spec = pl.BlockSpec((2, ROWBLK, HALF), lambda i: (0, i, 0))

_tca = pl.pallas_call(
    _tca_body,
    grid=(_GRID,),
    in_specs=[_row_spec, _w_spec, _degp_spec],
    out_specs=[_half2_spec, _row_spec, _dinv_spec],
    out_shape=[
        jax.ShapeDtypeStruct((2, N, HALF), jnp.float32),
        jax.ShapeDtypeStruct((N, D), jnp.float32),
        jax.ShapeDtypeStruct((N, HALF), jnp.float32),
    ],
)

_tcb = pl.pallas_call(
    _tcb_body,
    grid=(_GRID,),
    in_specs=[_half2_spec, _row_spec, _dinv_spec, _w_spec, _b_spec],
    out_specs=[_half2_spec, _row_spec],
    out_shape=[
        jax.ShapeDtypeStruct((2, N, HALF), jnp.float32),
        jax.ShapeDtypeStruct((N, D), jnp.float32),
    ],
)

_tcc = pl.pallas_call(
    _tcc_body,
    grid=(_GRID,),
    in_specs=[_half2_spec, _row_spec, _dinv_spec, _b_spec],
    out_specs=_row_spec,
    out_shape=jax.ShapeDtypeStruct((N, D), jnp.float32),
)


@jax.jit
def kernel(x, edge_index, W1, b1, W2, b2):
    src = edge_index[0].astype(jnp.int32)
    dst = edge_index[1].astype(jnp.int32)
    b1r = b1.reshape(1, D)
    b2r = b2.reshape(1, D)

    degp = _deg_kernel(dst).reshape(2, N, HALF)
    hhat, self1, dinv = _tca(x, W1, degp)
    agg1 = _agg_kernel(hhat.reshape(NCORE * N, HALF), src, dst)
    hhat2, self2 = _tcb(agg1.reshape(2, N, HALF), self1, dinv, W2, b1r)
    agg2 = _agg_kernel(hhat2.reshape(NCORE * N, HALF), src, dst)
    return _tcc(agg2.reshape(2, N, HALF), self2, dinv, b2r)


# speedup vs baseline: 13.8662x; 1.4499x over previous
import functools

import jax
import jax.numpy as jnp
from jax import lax
from jax.experimental import pallas as pl
from jax.experimental.pallas import tpu as pltpu
import jax.experimental.pallas.tpu_sc as plsc

N = 10000
E = 160000
D = 256
HALF = 128
CH = 128
NCHUNK = E // CH
NCORE = 2
NSUB = 16
BLK = 80
NBLK = N // BLK
ROWBLK = 1000

_mesh = plsc.VectorSubcoreMesh(core_axis_name="c", subcore_axis_name="s")


def _zero_fill(ref, rows, cols):
    per_row = cols // 16

    def body(k, _):
        ref[k // per_row, pl.ds((k % per_row) * 16, 16)] = jnp.zeros(
            (16,), jnp.float32)
        return 0

    lax.fori_loop(0, rows * per_row, body, 0)


def _nsplit(total, s):
    return jnp.where(s < (total % NSUB), total // NSUB + 1, total // NSUB)


@functools.partial(
    pl.kernel,
    out_type=jax.ShapeDtypeStruct((NCORE * N, HALF), jnp.float32),
    mesh=_mesh,
    scratch_types=[
        pltpu.VMEM((CH,), jnp.int32),
        pltpu.VMEM((CH, HALF), jnp.float32),
        pltpu.VMEM((BLK, HALF), jnp.float32),
        pltpu.VMEM_SHARED((N, HALF), jnp.float32),
    ],
)
def _deg_kernel(dst_hbm, out_hbm, dst_v, ones_v, z_v, acc_sh):
    c = lax.axis_index("c")
    s = lax.axis_index("s")
    _zero_fill(z_v, BLK, HALF)

    def ones_body(k, _):
        ones_v[k // 8, pl.ds((k % 8) * 16, 16)] = jnp.ones((16,), jnp.float32)
        return 0

    lax.fori_loop(0, CH * 8, ones_body, 0)

    def zero_acc(k, _):
        off = pl.multiple_of((k * NSUB + s) * BLK, 8)
        pltpu.sync_copy(z_v, acc_sh.at[pl.ds(off, BLK)])
        return 0

    lax.fori_loop(0, _nsplit(NBLK, s), zero_acc, 0)
    plsc.subcore_barrier()

    half = NCHUNK // NCORE

    def body(k, _):
        off = pl.multiple_of((c * half + k * NSUB + s) * CH, 8)
        pltpu.sync_copy(dst_hbm.at[pl.ds(off, CH)], dst_v)
        pltpu.sync_copy(ones_v, acc_sh.at[dst_v], add=True)
        return 0

    lax.fori_loop(0, _nsplit(half, s), body, 0)
    plsc.subcore_barrier()

    def wout(k, _):
        off = pl.multiple_of((k * NSUB + s) * BLK, 8)
        dst_off = pl.multiple_of(c * N + (k * NSUB + s) * BLK, 8)
        pltpu.sync_copy(acc_sh.at[pl.ds(off, BLK)],
                        out_hbm.at[pl.ds(dst_off, BLK)])
        return 0

    lax.fori_loop(0, _nsplit(NBLK, s), wout, 0)


_NBUF = 2


@functools.partial(
    pl.kernel,
    out_type=jax.ShapeDtypeStruct((NCORE * N, HALF), jnp.float32),
    mesh=_mesh,
    scratch_types=[
        [pltpu.VMEM((CH,), jnp.int32)] * _NBUF,
        [pltpu.VMEM((CH,), jnp.int32)] * _NBUF,
        [pltpu.VMEM((CH, HALF), jnp.float32)] * _NBUF,
        pltpu.VMEM((BLK, HALF), jnp.float32),
        pltpu.VMEM_SHARED((N, HALF), jnp.float32),
        [pltpu.SemaphoreType.DMA] * _NBUF,
    ],
)
def _agg_kernel(tab_hbm, src_hbm, dst_hbm, out_hbm,
                src_vs, dst_vs, rows_vs, z_v, acc_sh, sems):
    c = lax.axis_index("c")
    s = lax.axis_index("s")
    _zero_fill(z_v, BLK, HALF)

    def zero_acc(k, _):
        off = pl.multiple_of((k * NSUB + s) * BLK, 8)
        pltpu.sync_copy(z_v, acc_sh.at[pl.ds(off, BLK)])
        return 0

    lax.fori_loop(0, _nsplit(NBLK, s), zero_acc, 0)
    plsc.subcore_barrier()

    base = c * N
    n = _nsplit(NCHUNK, s)

    def load_and_fire(item, sv, dv, rv, sem):
        off = pl.multiple_of((item * NSUB + s) * CH, 8)
        pltpu.sync_copy(src_hbm.at[pl.ds(off, CH)], sv)
        pltpu.sync_copy(dst_hbm.at[pl.ds(off, CH)], dv)

        def shift(j, _):
            sv[pl.ds(j * 16, 16)] = sv[pl.ds(j * 16, 16)] + base
            return 0

        lax.fori_loop(0, CH // 16, shift, 0)
        pltpu.async_copy(tab_hbm.at[sv], rv, sem)

    for b in range(_NBUF):
        load_and_fire(jnp.int32(b), src_vs[b], dst_vs[b], rows_vs[b],
                      sems[b])

    def body(k, _):
        for b in range(_NBUF):
            item = _NBUF * k + b
            sv, dv, rv, sem = (src_vs[b], dst_vs[b], rows_vs[b], sems[b])

            def step(sv=sv, dv=dv, rv=rv, sem=sem, item=item):
                pltpu.make_async_copy(tab_hbm.at[sv], rv, sem).wait()
                pltpu.sync_copy(rv, acc_sh.at[dv], add=True)

                def refill():
                    load_and_fire(item + _NBUF, sv, dv, rv, sem)

                pl.when(item + _NBUF < n)(refill)

            pl.when(item < n)(step)

        return 0

    max_n = NCHUNK // NSUB + 1
    lax.fori_loop(0, (max_n + _NBUF - 1) // _NBUF, body, 0)
    plsc.subcore_barrier()

    def wout(k, _):
        off = pl.multiple_of((k * NSUB + s) * BLK, 8)
        dst_off = pl.multiple_of(c * N + (k * NSUB + s) * BLK, 8)
        pltpu.sync_copy(acc_sh.at[pl.ds(off, BLK)],
                        out_hbm.at[pl.ds(dst_off, BLK)])
        return 0

    lax.fori_loop(0, _nsplit(NBLK, s), wout, 0)


def _tca_body(x_ref, w_ref, degp_ref, hhat_ref, self_ref, dinv_ref):
    h = jnp.dot(x_ref[...], w_ref[...], preferred_element_type=jnp.float32)
    degp = degp_ref[...]
    deg = degp[0, :, 0] + degp[1, :, 0] + 1.0
    dinv = lax.rsqrt(deg)[:, None]
    hh = h * dinv
    hhat_ref[0] = hh[:, :HALF]
    hhat_ref[1] = hh[:, HALF:]
    self_ref[...] = hh * dinv
    dinv_ref[...] = jnp.broadcast_to(dinv, (ROWBLK, HALF))


def _tcb_body(agg_ref, self_ref, dinv_ref, w_ref, b_ref, hhat_ref, self2_ref):
    agg = jnp.concatenate([agg_ref[0], agg_ref[1]], axis=1)
    dinv = dinv_ref[:, :1]
    u = jnp.maximum(agg * dinv + self_ref[...] + b_ref[...], 0.0)
    h2 = jnp.dot(u, w_ref[...], preferred_element_type=jnp.float32)
    hh2 = h2 * dinv
    hhat_ref[0] = hh2[:, :HALF]
    hhat_ref[1] = hh2[:, HALF:]
    self2_ref[...] = hh2 * dinv


def _tcc_body(agg_ref, self_ref, dinv_ref, b_ref, out_ref):
    agg = jnp.concatenate([agg_ref[0], agg_ref[1]], axis=1)
    dinv = dinv_ref[:, :1]
    out_ref[...] = agg * dinv + self_ref[...] + b_ref[...]


_GRID = N // ROWBLK

_row_spec = pl.BlockSpec((ROWBLK, D), lambda i: (i, 0))
_half2_spec = pl.BlockSpec((2, ROWBLK, HALF), lambda i: (0, i, 0))
_dinv_spec = pl.BlockSpec((ROWBLK, HALF), lambda i: (i, 0))
_w_spec = pl.BlockSpec((D, D), lambda i: (0, 0))
_b_spec = pl.BlockSpec((1, D), lambda i: (0, 0))
_degp_spec = pl.BlockSpec((2, ROWBLK, HALF), lambda i: (0, i, 0))

_tca = pl.pallas_call(
    _tca_body,
    grid=(_GRID,),
    in_specs=[_row_spec, _w_spec, _degp_spec],
    out_specs=[_half2_spec, _row_spec, _dinv_spec],
    out_shape=[
        jax.ShapeDtypeStruct((2, N, HALF), jnp.float32),
        jax.ShapeDtypeStruct((N, D), jnp.float32),
        jax.ShapeDtypeStruct((N, HALF), jnp.float32),
    ],
)

_tcb = pl.pallas_call(
    _tcb_body,
    grid=(_GRID,),
    in_specs=[_half2_spec, _row_spec, _dinv_spec, _w_spec, _b_spec],
    out_specs=[_half2_spec, _row_spec],
    out_shape=[
        jax.ShapeDtypeStruct((2, N, HALF), jnp.float32),
        jax.ShapeDtypeStruct((N, D), jnp.float32),
    ],
)

_tcc = pl.pallas_call(
    _tcc_body,
    grid=(_GRID,),
    in_specs=[_half2_spec, _row_spec, _dinv_spec, _b_spec],
    out_specs=_row_spec,
    out_shape=jax.ShapeDtypeStruct((N, D), jnp.float32),
)


@jax.jit
def kernel(x, edge_index, W1, b1, W2, b2):
    src = edge_index[0].astype(jnp.int32)
    dst = edge_index[1].astype(jnp.int32)
    b1r = b1.reshape(1, D)
    b2r = b2.reshape(1, D)

    degp = _deg_kernel(dst).reshape(2, N, HALF)
    hhat, self1, dinv = _tca(x, W1, degp)
    agg1 = _agg_kernel(hhat.reshape(NCORE * N, HALF), src, dst)
    hhat2, self2 = _tcb(agg1.reshape(2, N, HALF), self1, dinv, W2, b1r)
    agg2 = _agg_kernel(hhat2.reshape(NCORE * N, HALF), src, dst)
    return _tcc(agg2.reshape(2, N, HALF), self2, dinv, b2r)


# speedup vs baseline: 16.7462x; 1.2077x over previous
import functools

import jax
import jax.numpy as jnp
from jax import lax
from jax.experimental import pallas as pl
from jax.experimental.pallas import tpu as pltpu
import jax.experimental.pallas.tpu_sc as plsc

N = 10000
E = 160000
D = 256
HALF = 128
CH = 128
NCHUNK = E // CH
NCORE = 2
NSUB = 16
BLK = 80
NBLK = N // BLK
ROWBLK = 1000

_mesh = plsc.VectorSubcoreMesh(core_axis_name="c", subcore_axis_name="s")


def _zero_fill(ref, rows, cols):
    per_row = cols // 16

    def body(k, _):
        ref[k // per_row, pl.ds((k % per_row) * 16, 16)] = jnp.zeros(
            (16,), jnp.float32)
        return 0

    lax.fori_loop(0, rows * per_row, body, 0)


def _nsplit(total, s):
    return jnp.where(s < (total % NSUB), total // NSUB + 1, total // NSUB)


_DEG_W = HALF
_DNB = 3


@functools.partial(
    pl.kernel,
    out_type=jax.ShapeDtypeStruct((NCORE * N, _DEG_W), jnp.float32),
    mesh=_mesh,
    scratch_types=[
        [pltpu.VMEM((CH,), jnp.int32)] * _DNB,
        pltpu.VMEM((CH, HALF), jnp.float32),
        pltpu.VMEM((BLK, HALF), jnp.float32),
        pltpu.VMEM_SHARED((N, HALF), jnp.float32),
        [pltpu.SemaphoreType.DMA] * _DNB,
        [pltpu.SemaphoreType.DMA] * _DNB,
    ],
)
def _deg_kernel(dst_hbm, out_hbm, dst_vs, ones_v, z_v, acc_sh,
                isems, ssems):
    c = lax.axis_index("c")
    s = lax.axis_index("s")
    _zero_fill(z_v, BLK, HALF)

    def ones_body(k, _):
        ones_v[k // 8, pl.ds((k % 8) * 16, 16)] = jnp.ones((16,), jnp.float32)
        return 0

    lax.fori_loop(0, CH * 8, ones_body, 0)

    def zero_acc(k, _):
        off = pl.multiple_of((k * NSUB + s) * BLK, 8)
        pltpu.sync_copy(z_v, acc_sh.at[pl.ds(off, BLK)])
        return 0

    lax.fori_loop(0, _nsplit(NBLK, s), zero_acc, 0)
    plsc.subcore_barrier()

    half = NCHUNK // NCORE
    n = _nsplit(half, s)

    def fire_idx(item, b):
        off = pl.multiple_of((c * half + item * NSUB + s) * CH, 8)
        pltpu.async_copy(dst_hbm.at[pl.ds(off, CH)], dst_vs[b], isems[b])

    for b in range(_DNB):
        pl.when(b < n)(lambda b=b: fire_idx(jnp.int32(b), b))

    def body(k, _):
        for b in range(_DNB):
            item = _DNB * k + b

            def step(b=b, item=item):
                pltpu.make_async_copy(
                    dst_hbm.at[pl.ds(0, CH)], dst_vs[b], isems[b]).wait()
                pltpu.async_copy(
                    ones_v, acc_sh.at[dst_vs[b]], ssems[b], add=True)

                def refill():
                    pltpu.make_async_copy(
                        ones_v, acc_sh.at[dst_vs[b]], ssems[b]).wait()
                    fire_idx(item + _DNB, b)

                pl.when(item + _DNB < n)(refill)

            pl.when(item < n)(step)
        return 0

    max_n = half // NSUB + 1
    lax.fori_loop(0, (max_n + _DNB - 1) // _DNB, body, 0)
    for b in range(_DNB):
        def drain(b=b, nn=n):
            pltpu.make_async_copy(ones_v, acc_sh.at[dst_vs[b]],
                                  ssems[b]).wait()
        pl.when(jnp.int32(b) < jnp.minimum(n, _DNB))(drain)
    plsc.subcore_barrier()

    def wout(k, _):
        off = pl.multiple_of((k * NSUB + s) * BLK, 8)
        dst_off = pl.multiple_of(c * N + (k * NSUB + s) * BLK, 8)
        pltpu.sync_copy(acc_sh.at[pl.ds(off, BLK)],
                        out_hbm.at[pl.ds(dst_off, BLK)])
        return 0

    lax.fori_loop(0, _nsplit(NBLK, s), wout, 0)


_NBUF = 2


@functools.partial(
    pl.kernel,
    out_type=jax.ShapeDtypeStruct((NCORE * N, HALF), jnp.float32),
    mesh=_mesh,
    scratch_types=[
        [pltpu.VMEM((CH,), jnp.int32)] * _NBUF,
        [pltpu.VMEM((CH,), jnp.int32)] * _NBUF,
        [pltpu.VMEM((CH, HALF), jnp.float32)] * _NBUF,
        pltpu.VMEM((BLK, HALF), jnp.float32),
        pltpu.VMEM_SHARED((N, HALF), jnp.float32),
        [pltpu.SemaphoreType.DMA] * _NBUF,
        [pltpu.SemaphoreType.DMA] * _NBUF,
    ],
)
def _agg_kernel(tab_hbm, src_hbm, dst_hbm, out_hbm,
                src_vs, dst_vs, rows_vs, z_v, acc_sh, gsems, isems):
    c = lax.axis_index("c")
    s = lax.axis_index("s")
    _zero_fill(z_v, BLK, HALF)

    def zero_acc(k, _):
        off = pl.multiple_of((k * NSUB + s) * BLK, 8)
        pltpu.sync_copy(z_v, acc_sh.at[pl.ds(off, BLK)])
        return 0

    lax.fori_loop(0, _nsplit(NBLK, s), zero_acc, 0)
    plsc.subcore_barrier()

    base = c * N
    n = _nsplit(NCHUNK, s)

    def fire_idx(item, b):
        off = pl.multiple_of((item * NSUB + s) * CH, 8)
        pltpu.async_copy(src_hbm.at[pl.ds(off, CH)], src_vs[b], isems[b])
        pltpu.async_copy(dst_hbm.at[pl.ds(off, CH)], dst_vs[b], isems[b])

    def wait_idx_fire_gather(b):
        pltpu.make_async_copy(
            src_hbm.at[pl.ds(0, CH)], src_vs[b], isems[b]).wait()
        pltpu.make_async_copy(
            dst_hbm.at[pl.ds(0, CH)], dst_vs[b], isems[b]).wait()

        def shift(j, _):
            src_vs[b][pl.ds(j * 16, 16)] = src_vs[b][pl.ds(j * 16, 16)] + base
            return 0

        lax.fori_loop(0, CH // 16, shift, 0)
        pltpu.async_copy(tab_hbm.at[src_vs[b]], rows_vs[b], gsems[b])

    fire_idx(jnp.int32(0), 0)
    fire_idx(jnp.int32(1), 1)
    wait_idx_fire_gather(0)

    def body(k, _):
        for b in range(_NBUF):
            item = _NBUF * k + b

            def step(b=b, item=item):
                nb = (b + 1) % _NBUF
                pl.when(item + 1 < n)(lambda: wait_idx_fire_gather(nb))
                pltpu.make_async_copy(
                    tab_hbm.at[src_vs[b]], rows_vs[b], gsems[b]).wait()
                pltpu.sync_copy(rows_vs[b], acc_sh.at[dst_vs[b]], add=True)
                pl.when(item + _NBUF < n)(
                    lambda: fire_idx(item + _NBUF, b))

            pl.when(item < n)(step)
        return 0

    max_n = NCHUNK // NSUB + 1
    lax.fori_loop(0, (max_n + _NBUF - 1) // _NBUF, body, 0)
    plsc.subcore_barrier()

    def wout(k, _):
        off = pl.multiple_of((k * NSUB + s) * BLK, 8)
        dst_off = pl.multiple_of(c * N + (k * NSUB + s) * BLK, 8)
        pltpu.sync_copy(acc_sh.at[pl.ds(off, BLK)],
                        out_hbm.at[pl.ds(dst_off, BLK)])
        return 0

    lax.fori_loop(0, _nsplit(NBLK, s), wout, 0)


def _tca_body(x_ref, w_ref, degp_ref, hhat_ref, dinv_ref):
    h = jnp.dot(x_ref[...], w_ref[...], preferred_element_type=jnp.float32)
    degp = degp_ref[...]
    deg = degp[0, :, 0] + degp[1, :, 0] + 1.0
    dinv = lax.rsqrt(deg)[:, None]
    hh = h * dinv
    hhat_ref[0] = hh[:, :HALF]
    hhat_ref[1] = hh[:, HALF:]
    dinv_ref[...] = jnp.broadcast_to(dinv, (ROWBLK, HALF))


def _tcb_body(agg_ref, hhat_ref, dinv_ref, w_ref, b_ref, hhat2_ref):
    agg = jnp.concatenate([agg_ref[0] + hhat_ref[0],
                           agg_ref[1] + hhat_ref[1]], axis=1)
    dinv = dinv_ref[:, :1]
    u = jnp.maximum(agg * dinv + b_ref[...], 0.0)
    h2 = jnp.dot(u, w_ref[...], preferred_element_type=jnp.float32)
    hh2 = h2 * dinv
    hhat2_ref[0] = hh2[:, :HALF]
    hhat2_ref[1] = hh2[:, HALF:]


def _tcc_body(agg_ref, hhat_ref, dinv_ref, b_ref, out_ref):
    agg = jnp.concatenate([agg_ref[0] + hhat_ref[0],
                           agg_ref[1] + hhat_ref[1]], axis=1)
    dinv = dinv_ref[:, :1]
    out_ref[...] = agg * dinv + b_ref[...]


_GRID = N // ROWBLK

_row_spec = pl.BlockSpec((ROWBLK, D), lambda i: (i, 0))
_half2_spec = pl.BlockSpec((2, ROWBLK, HALF), lambda i: (0, i, 0))
_dinv_spec = pl.BlockSpec((ROWBLK, HALF), lambda i: (i, 0))
_w_spec = pl.BlockSpec((D, D), lambda i: (0, 0))
_b_spec = pl.BlockSpec((1, D), lambda i: (0, 0))
_degp_spec = pl.BlockSpec((2, ROWBLK, _DEG_W), lambda i: (0, i, 0))

_tca = pl.pallas_call(
    _tca_body,
    grid=(_GRID,),
    in_specs=[_row_spec, _w_spec, _degp_spec],
    out_specs=[_half2_spec, _dinv_spec],
    out_shape=[
        jax.ShapeDtypeStruct((2, N, HALF), jnp.float32),
        jax.ShapeDtypeStruct((N, HALF), jnp.float32),
    ],
)

_tcb = pl.pallas_call(
    _tcb_body,
    grid=(_GRID,),
    in_specs=[_half2_spec, _half2_spec, _dinv_spec, _w_spec, _b_spec],
    out_specs=_half2_spec,
    out_shape=jax.ShapeDtypeStruct((2, N, HALF), jnp.float32),
)

_tcc = pl.pallas_call(
    _tcc_body,
    grid=(_GRID,),
    in_specs=[_half2_spec, _half2_spec, _dinv_spec, _b_spec],
    out_specs=_row_spec,
    out_shape=jax.ShapeDtypeStruct((N, D), jnp.float32),
)


@jax.jit
def kernel(x, edge_index, W1, b1, W2, b2):
    src = edge_index[0].astype(jnp.int32)
    dst = edge_index[1].astype(jnp.int32)
    b1r = b1.reshape(1, D)
    b2r = b2.reshape(1, D)

    degp = _deg_kernel(dst).reshape(2, N, _DEG_W)
    hhat, dinv = _tca(x, W1, degp)
    agg1 = _agg_kernel(hhat.reshape(NCORE * N, HALF), src, dst)
    hhat2 = _tcb(agg1.reshape(2, N, HALF), hhat, dinv, W2, b1r)
    agg2 = _agg_kernel(hhat2.reshape(NCORE * N, HALF), src, dst)
    return _tcc(agg2.reshape(2, N, HALF), hhat2, dinv, b2r)


# speedup vs baseline: 17.2281x; 1.0288x over previous
import functools

import jax
import jax.numpy as jnp
from jax import lax
from jax.experimental import pallas as pl
from jax.experimental.pallas import tpu as pltpu
import jax.experimental.pallas.tpu_sc as plsc

N = 10000
E = 160000
D = 256
HALF = 128
CH = 128
NCHUNK = E // CH
NCORE = 2
NSUB = 16
BLK = 80
NBLK = N // BLK
ROWBLK = 1000

_mesh = plsc.VectorSubcoreMesh(core_axis_name="c", subcore_axis_name="s")


def _zero_fill(ref, rows, cols):
    per_row = cols // 16

    def body(k, _):
        ref[k // per_row, pl.ds((k % per_row) * 16, 16)] = jnp.zeros(
            (16,), jnp.float32)
        return 0

    lax.fori_loop(0, rows * per_row, body, 0)


def _nsplit(total, s):
    return jnp.where(s < (total % NSUB), total // NSUB + 1, total // NSUB)


_DEG_W = HALF
_DNB = 3


@functools.partial(
    pl.kernel,
    out_type=jax.ShapeDtypeStruct((NCORE * N, _DEG_W), jnp.float32),
    mesh=_mesh,
    scratch_types=[
        [pltpu.VMEM((CH,), jnp.int32)] * _DNB,
        pltpu.VMEM((CH, HALF), jnp.float32),
        pltpu.VMEM((BLK, HALF), jnp.float32),
        pltpu.VMEM_SHARED((N, HALF), jnp.float32),
        [pltpu.SemaphoreType.DMA] * _DNB,
        [pltpu.SemaphoreType.DMA] * _DNB,
    ],
)
def _deg_kernel(dst_hbm, out_hbm, dst_vs, ones_v, z_v, acc_sh,
                isems, ssems):
    c = lax.axis_index("c")
    s = lax.axis_index("s")
    _zero_fill(z_v, BLK, HALF)

    def ones_body(k, _):
        ones_v[k // 8, pl.ds((k % 8) * 16, 16)] = jnp.ones((16,), jnp.float32)
        return 0

    lax.fori_loop(0, CH * 8, ones_body, 0)

    def zero_acc(k, _):
        off = pl.multiple_of((k * NSUB + s) * BLK, 8)
        pltpu.sync_copy(z_v, acc_sh.at[pl.ds(off, BLK)])
        return 0

    lax.fori_loop(0, _nsplit(NBLK, s), zero_acc, 0)
    plsc.subcore_barrier()

    half = NCHUNK // NCORE
    n = _nsplit(half, s)

    def fire_idx(item, b):
        off = pl.multiple_of((c * half + item * NSUB + s) * CH, 8)
        pltpu.async_copy(dst_hbm.at[pl.ds(off, CH)], dst_vs[b], isems[b])

    for b in range(_DNB):
        pl.when(b < n)(lambda b=b: fire_idx(jnp.int32(b), b))

    def body(k, _):
        for b in range(_DNB):
            item = _DNB * k + b

            def step(b=b, item=item):
                pltpu.make_async_copy(
                    dst_hbm.at[pl.ds(0, CH)], dst_vs[b], isems[b]).wait()
                pltpu.async_copy(
                    ones_v, acc_sh.at[dst_vs[b]], ssems[b], add=True)

                def refill():
                    pltpu.make_async_copy(
                        ones_v, acc_sh.at[dst_vs[b]], ssems[b]).wait()
                    fire_idx(item + _DNB, b)

                pl.when(item + _DNB < n)(refill)

            pl.when(item < n)(step)
        return 0

    max_n = half // NSUB + 1
    lax.fori_loop(0, (max_n + _DNB - 1) // _DNB, body, 0)
    for b in range(_DNB):
        def drain(b=b, nn=n):
            pltpu.make_async_copy(ones_v, acc_sh.at[dst_vs[b]],
                                  ssems[b]).wait()
        pl.when(jnp.int32(b) < jnp.minimum(n, _DNB))(drain)
    plsc.subcore_barrier()

    def wout(k, _):
        off = pl.multiple_of((k * NSUB + s) * BLK, 8)
        dst_off = pl.multiple_of(c * N + (k * NSUB + s) * BLK, 8)
        pltpu.sync_copy(acc_sh.at[pl.ds(off, BLK)],
                        out_hbm.at[pl.ds(dst_off, BLK)])
        return 0

    lax.fori_loop(0, _nsplit(NBLK, s), wout, 0)


_NBUF = 3


@functools.partial(
    pl.kernel,
    out_type=jax.ShapeDtypeStruct((NCORE * N, HALF), jnp.float32),
    mesh=_mesh,
    scratch_types=[
        [pltpu.VMEM((CH,), jnp.int32)] * _NBUF,
        [pltpu.VMEM((CH,), jnp.int32)] * _NBUF,
        [pltpu.VMEM((CH, HALF), jnp.float32)] * _NBUF,
        pltpu.VMEM_SHARED((N, HALF), jnp.float32),
        [pltpu.SemaphoreType.DMA] * _NBUF,
        [pltpu.SemaphoreType.DMA] * _NBUF,
    ],
)
def _agg_kernel(tab_hbm, src_hbm, dst_hbm, out_hbm,
                src_vs, dst_vs, rows_vs, acc_sh, gsems, isems):
    c = lax.axis_index("c")
    s = lax.axis_index("s")
    _zero_fill(rows_vs[0], BLK, HALF)

    def zero_acc(k, _):
        off = pl.multiple_of((k * NSUB + s) * BLK, 8)
        pltpu.sync_copy(rows_vs[0].at[pl.ds(0, BLK)],
                        acc_sh.at[pl.ds(off, BLK)])
        return 0

    lax.fori_loop(0, _nsplit(NBLK, s), zero_acc, 0)
    plsc.subcore_barrier()

    base = c * N
    n = _nsplit(NCHUNK, s)

    def fire_idx(item, b):
        off = pl.multiple_of((item * NSUB + s) * CH, 8)
        pltpu.async_copy(src_hbm.at[pl.ds(off, CH)], src_vs[b], isems[b])
        pltpu.async_copy(dst_hbm.at[pl.ds(off, CH)], dst_vs[b], isems[b])

    def wait_idx_fire_gather(b):
        pltpu.make_async_copy(
            src_hbm.at[pl.ds(0, CH)], src_vs[b], isems[b]).wait()
        pltpu.make_async_copy(
            dst_hbm.at[pl.ds(0, CH)], dst_vs[b], isems[b]).wait()

        def shift(j, _):
            src_vs[b][pl.ds(j * 16, 16)] = src_vs[b][pl.ds(j * 16, 16)] + base
            return 0

        lax.fori_loop(0, CH // 16, shift, 0)
        pltpu.async_copy(tab_hbm.at[src_vs[b]], rows_vs[b], gsems[b])

    for b in range(_NBUF):
        fire_idx(jnp.int32(b), b)
    for b in range(_NBUF - 1):
        wait_idx_fire_gather(b)

    def body(k, _):
        for b in range(_NBUF):
            item = _NBUF * k + b

            def step(b=b, item=item):
                nb = (b + _NBUF - 1) % _NBUF
                pl.when(item + _NBUF - 1 < n)(
                    lambda: wait_idx_fire_gather(nb))
                pltpu.make_async_copy(
                    tab_hbm.at[src_vs[b]], rows_vs[b], gsems[b]).wait()
                pltpu.sync_copy(rows_vs[b], acc_sh.at[dst_vs[b]], add=True)
                pl.when(item + _NBUF < n)(
                    lambda: fire_idx(item + _NBUF, b))

            pl.when(item < n)(step)
        return 0

    max_n = NCHUNK // NSUB + 1
    lax.fori_loop(0, (max_n + _NBUF - 1) // _NBUF, body, 0)
    plsc.subcore_barrier()

    def wout(k, _):
        off = pl.multiple_of((k * NSUB + s) * BLK, 8)
        dst_off = pl.multiple_of(c * N + (k * NSUB + s) * BLK, 8)
        pltpu.sync_copy(acc_sh.at[pl.ds(off, BLK)],
                        out_hbm.at[pl.ds(dst_off, BLK)])
        return 0

    lax.fori_loop(0, _nsplit(NBLK, s), wout, 0)


def _tca_body(x_ref, w_ref, degp_ref, hhat_ref, dinv_ref):
    h = jnp.dot(x_ref[...], w_ref[...], preferred_element_type=jnp.float32)
    degp = degp_ref[...]
    deg = degp[0, :, 0] + degp[1, :, 0] + 1.0
    dinv = lax.rsqrt(deg)[:, None]
    hh = h * dinv
    hhat_ref[0] = hh[:, :HALF]
    hhat_ref[1] = hh[:, HALF:]
    dinv_ref[...] = jnp.broadcast_to(dinv, (ROWBLK, HALF))


def _tcb_body(agg_ref, hhat_ref, dinv_ref, w_ref, b_ref, hhat2_ref):
    agg = jnp.concatenate([agg_ref[0] + hhat_ref[0],
                           agg_ref[1] + hhat_ref[1]], axis=1)
    dinv = dinv_ref[:, :1]
    u = jnp.maximum(agg * dinv + b_ref[...], 0.0)
    h2 = jnp.dot(u, w_ref[...], preferred_element_type=jnp.float32)
    hh2 = h2 * dinv
    hhat2_ref[0] = hh2[:, :HALF]
    hhat2_ref[1] = hh2[:, HALF:]


def _tcc_body(agg_ref, hhat_ref, dinv_ref, b_ref, out_ref):
    agg = jnp.concatenate([agg_ref[0] + hhat_ref[0],
                           agg_ref[1] + hhat_ref[1]], axis=1)
    dinv = dinv_ref[:, :1]
    out_ref[...] = agg * dinv + b_ref[...]


_GRID = N // ROWBLK

_row_spec = pl.BlockSpec((ROWBLK, D), lambda i: (i, 0))
_half2_spec = pl.BlockSpec((2, ROWBLK, HALF), lambda i: (0, i, 0))
_dinv_spec = pl.BlockSpec((ROWBLK, HALF), lambda i: (i, 0))
_w_spec = pl.BlockSpec((D, D), lambda i: (0, 0))
_b_spec = pl.BlockSpec((1, D), lambda i: (0, 0))
_degp_spec = pl.BlockSpec((2, ROWBLK, _DEG_W), lambda i: (0, i, 0))

_tca = pl.pallas_call(
    _tca_body,
    grid=(_GRID,),
    in_specs=[_row_spec, _w_spec, _degp_spec],
    out_specs=[_half2_spec, _dinv_spec],
    out_shape=[
        jax.ShapeDtypeStruct((2, N, HALF), jnp.float32),
        jax.ShapeDtypeStruct((N, HALF), jnp.float32),
    ],
)

_tcb = pl.pallas_call(
    _tcb_body,
    grid=(_GRID,),
    in_specs=[_half2_spec, _half2_spec, _dinv_spec, _w_spec, _b_spec],
    out_specs=_half2_spec,
    out_shape=jax.ShapeDtypeStruct((2, N, HALF), jnp.float32),
)

_tcc = pl.pallas_call(
    _tcc_body,
    grid=(_GRID,),
    in_specs=[_half2_spec, _half2_spec, _dinv_spec, _b_spec],
    out_specs=_row_spec,
    out_shape=jax.ShapeDtypeStruct((N, D), jnp.float32),
)


@jax.jit
def kernel(x, edge_index, W1, b1, W2, b2):
    src = edge_index[0].astype(jnp.int32)
    dst = edge_index[1].astype(jnp.int32)
    b1r = b1.reshape(1, D)
    b2r = b2.reshape(1, D)

    degp = _deg_kernel(dst).reshape(2, N, _DEG_W)
    hhat, dinv = _tca(x, W1, degp)
    agg1 = _agg_kernel(hhat.reshape(NCORE * N, HALF), src, dst)
    hhat2 = _tcb(agg1.reshape(2, N, HALF), hhat, dinv, W2, b1r)
    agg2 = _agg_kernel(hhat2.reshape(NCORE * N, HALF), src, dst)
    return _tcc(agg2.reshape(2, N, HALF), hhat2, dinv, b2r)


# speedup vs baseline: 18.4815x; 1.0728x over previous
import functools

import jax
import jax.numpy as jnp
from jax import lax
from jax.experimental import pallas as pl
from jax.experimental.pallas import tpu as pltpu
import jax.experimental.pallas.tpu_sc as plsc

N = 10000
E = 160000
D = 256
HALF = 128
CH = 128
NCHUNK = E // CH
NCORE = 2
NSUB = 16
BLK = 80
NBLK = N // BLK
ROWBLK = 1000

_mesh = plsc.VectorSubcoreMesh(core_axis_name="c", subcore_axis_name="s")


def _zero_fill(ref, rows, cols):
    per_row = cols // 16

    def body(k, _):
        ref[k // per_row, pl.ds((k % per_row) * 16, 16)] = jnp.zeros(
            (16,), jnp.float32)
        return 0

    lax.fori_loop(0, rows * per_row, body, 0)


def _nsplit(total, s):
    return jnp.where(s < (total % NSUB), total // NSUB + 1, total // NSUB)


_DEG_W = HALF
_DNB = 3


@functools.partial(
    pl.kernel,
    out_type=jax.ShapeDtypeStruct((NCORE * N, _DEG_W), jnp.float32),
    mesh=_mesh,
    scratch_types=[
        [pltpu.VMEM((CH,), jnp.int32)] * _DNB,
        pltpu.VMEM((CH, HALF), jnp.float32),
        pltpu.VMEM((BLK, HALF), jnp.float32),
        pltpu.VMEM_SHARED((N, HALF), jnp.float32),
        [pltpu.SemaphoreType.DMA] * _DNB,
        [pltpu.SemaphoreType.DMA] * _DNB,
    ],
)
def _deg_kernel(dst_hbm, out_hbm, dst_vs, ones_v, z_v, acc_sh,
                isems, ssems):
    c = lax.axis_index("c")
    s = lax.axis_index("s")
    _zero_fill(z_v, BLK, HALF)

    def ones_body(k, _):
        ones_v[k // 8, pl.ds((k % 8) * 16, 16)] = jnp.ones((16,), jnp.float32)
        return 0

    lax.fori_loop(0, CH * 8, ones_body, 0)

    def zero_acc(k, _):
        off = pl.multiple_of((k * NSUB + s) * BLK, 8)
        pltpu.sync_copy(z_v, acc_sh.at[pl.ds(off, BLK)])
        return 0

    lax.fori_loop(0, _nsplit(NBLK, s), zero_acc, 0)
    plsc.subcore_barrier()

    half = NCHUNK // NCORE
    n = _nsplit(half, s)

    def fire_idx(item, b):
        off = pl.multiple_of((c * half + item * NSUB + s) * CH, 8)
        pltpu.async_copy(dst_hbm.at[pl.ds(off, CH)], dst_vs[b], isems[b])

    for b in range(_DNB):
        pl.when(b < n)(lambda b=b: fire_idx(jnp.int32(b), b))

    def body(k, _):
        for b in range(_DNB):
            item = _DNB * k + b

            def step(b=b, item=item):
                pltpu.make_async_copy(
                    dst_hbm.at[pl.ds(0, CH)], dst_vs[b], isems[b]).wait()
                pltpu.async_copy(
                    ones_v, acc_sh.at[dst_vs[b]], ssems[b], add=True)

                def refill():
                    pltpu.make_async_copy(
                        ones_v, acc_sh.at[dst_vs[b]], ssems[b]).wait()
                    fire_idx(item + _DNB, b)

                pl.when(item + _DNB < n)(refill)

            pl.when(item < n)(step)
        return 0

    max_n = half // NSUB + 1
    lax.fori_loop(0, (max_n + _DNB - 1) // _DNB, body, 0)
    for b in range(_DNB):
        def drain(b=b, nn=n):
            pltpu.make_async_copy(ones_v, acc_sh.at[dst_vs[b]],
                                  ssems[b]).wait()
        pl.when(jnp.int32(b) < jnp.minimum(n, _DNB))(drain)
    plsc.subcore_barrier()

    def wout(k, _):
        off = pl.multiple_of((k * NSUB + s) * BLK, 8)
        dst_off = pl.multiple_of(c * N + (k * NSUB + s) * BLK, 8)
        pltpu.sync_copy(acc_sh.at[pl.ds(off, BLK)],
                        out_hbm.at[pl.ds(dst_off, BLK)])
        return 0

    lax.fori_loop(0, _nsplit(NBLK, s), wout, 0)


_NROW = 2
_NIDX = 4


@functools.partial(
    pl.kernel,
    out_type=jax.ShapeDtypeStruct((NCORE * N, HALF), jnp.float32),
    mesh=_mesh,
    scratch_types=[
        [pltpu.VMEM((CH,), jnp.int32)] * _NIDX,
        [pltpu.VMEM((CH,), jnp.int32)] * _NIDX,
        [pltpu.VMEM((CH, HALF), jnp.float32)] * _NROW,
        pltpu.VMEM_SHARED((N, HALF), jnp.float32),
        [pltpu.SemaphoreType.DMA] * _NROW,
        [pltpu.SemaphoreType.DMA] * _NROW,
        [pltpu.SemaphoreType.DMA] * _NIDX,
    ],
)
def _agg_kernel(tab_hbm, src_hbm, dst_hbm, out_hbm,
                src_vs, dst_vs, rows_vs, acc_sh, gsems, ssems, isems):
    c = lax.axis_index("c")
    s = lax.axis_index("s")
    _zero_fill(rows_vs[0], BLK, HALF)

    def zero_acc(k, _):
        off = pl.multiple_of((k * NSUB + s) * BLK, 8)
        pltpu.sync_copy(rows_vs[0].at[pl.ds(0, BLK)],
                        acc_sh.at[pl.ds(off, BLK)])
        return 0

    lax.fori_loop(0, _nsplit(NBLK, s), zero_acc, 0)
    plsc.subcore_barrier()

    base = c * N
    n = _nsplit(NCHUNK, s)

    def fire_idx(item, i4):
        off = pl.multiple_of((item * NSUB + s) * CH, 8)
        pltpu.async_copy(src_hbm.at[pl.ds(off, CH)], src_vs[i4], isems[i4])
        pltpu.async_copy(dst_hbm.at[pl.ds(off, CH)], dst_vs[i4], isems[i4])

    def fire_gather(i4, r2):
        pltpu.make_async_copy(
            src_hbm.at[pl.ds(0, CH)], src_vs[i4], isems[i4]).wait()
        pltpu.make_async_copy(
            src_hbm.at[pl.ds(0, CH)], dst_vs[i4], isems[i4]).wait()

        def shift(j, _):
            src_vs[i4][pl.ds(j * 16, 16)] = (
                src_vs[i4][pl.ds(j * 16, 16)] + base)
            return 0

        lax.fori_loop(0, CH // 16, shift, 0)
        pltpu.async_copy(tab_hbm.at[src_vs[i4]], rows_vs[r2], gsems[r2])

    def wait_gather_fire_scatter(i4, r2):
        pltpu.make_async_copy(
            tab_hbm.at[src_vs[i4]], rows_vs[r2], gsems[r2]).wait()
        pltpu.async_copy(rows_vs[r2], acc_sh.at[dst_vs[i4]], ssems[r2],
                         add=True)

    def wait_scatter(r2, i4):
        pltpu.make_async_copy(
            rows_vs[r2], acc_sh.at[dst_vs[i4]], ssems[r2]).wait()

    fire_idx(jnp.int32(0), 0)
    fire_idx(jnp.int32(1), 1)
    fire_idx(jnp.int32(2), 2)
    fire_gather(0, 0)
    fire_gather(1, 1)
    wait_gather_fire_scatter(0, 0)
    wait_scatter(0, 0)
    fire_idx(jnp.int32(3), 3)
    fire_idx(jnp.int32(4), 0)
    fire_gather(2, 0)
    wait_gather_fire_scatter(1, 1)

    def body(k, _):
        for jj in range(_NIDX):
            base_item = _NIDX * k + jj + 2
            i4 = (jj + 2) % _NIDX
            r2 = jj % _NROW

            def step(item=base_item, i4=i4, r2=r2):
                ni4 = (i4 + 1) % _NIDX
                nr2 = (r2 + 1) % _NROW

                def prep_next():
                    pi4 = (i4 + 3) % _NIDX
                    wait_scatter(nr2, pi4)
                    pl.when(item + 3 < n)(
                        lambda: fire_idx(item + 3, pi4))
                    fire_gather(ni4, nr2)

                pl.when(item + 1 < n)(prep_next)
                wait_gather_fire_scatter(i4, r2)

            pl.when(base_item < n)(step)
        return 0

    max_n = NCHUNK // NSUB + 1
    lax.fori_loop(0, (max_n - 2 + _NIDX - 1) // _NIDX, body, 0)

    for m in range(_NIDX):
        pl.when((n - 2) % _NIDX == m)(
            lambda m=m: wait_scatter(m % _NROW, m))
    for m in range(_NIDX):
        pl.when((n - 1) % _NIDX == m)(
            lambda m=m: wait_scatter(m % _NROW, m))
    plsc.subcore_barrier()

    def wout(k, _):
        off = pl.multiple_of((k * NSUB + s) * BLK, 8)
        dst_off = pl.multiple_of(c * N + (k * NSUB + s) * BLK, 8)
        pltpu.sync_copy(acc_sh.at[pl.ds(off, BLK)],
                        out_hbm.at[pl.ds(dst_off, BLK)])
        return 0

    lax.fori_loop(0, _nsplit(NBLK, s), wout, 0)


def _tca_body(x_ref, w_ref, degp_ref, hhat_ref, dinv_ref):
    h = jnp.dot(x_ref[...], w_ref[...], preferred_element_type=jnp.float32)
    degp = degp_ref[...]
    deg = degp[0, :, 0] + degp[1, :, 0] + 1.0
    dinv = lax.rsqrt(deg)[:, None]
    hh = h * dinv
    hhat_ref[0] = hh[:, :HALF]
    hhat_ref[1] = hh[:, HALF:]
    dinv_ref[...] = jnp.broadcast_to(dinv, (ROWBLK, HALF))


def _tcb_body(agg_ref, hhat_ref, dinv_ref, w_ref, b_ref, hhat2_ref):
    agg = jnp.concatenate([agg_ref[0] + hhat_ref[0],
                           agg_ref[1] + hhat_ref[1]], axis=1)
    dinv = dinv_ref[:, :1]
    u = jnp.maximum(agg * dinv + b_ref[...], 0.0)
    h2 = jnp.dot(u, w_ref[...], preferred_element_type=jnp.float32)
    hh2 = h2 * dinv
    hhat2_ref[0] = hh2[:, :HALF]
    hhat2_ref[1] = hh2[:, HALF:]


def _tcc_body(agg_ref, hhat_ref, dinv_ref, b_ref, out_ref):
    agg = jnp.concatenate([agg_ref[0] + hhat_ref[0],
                           agg_ref[1] + hhat_ref[1]], axis=1)
    dinv = dinv_ref[:, :1]
    out_ref[...] = agg * dinv + b_ref[...]


_GRID = N // ROWBLK

_row_spec = pl.BlockSpec((ROWBLK, D), lambda i: (i, 0))
_half2_spec = pl.BlockSpec((2, ROWBLK, HALF), lambda i: (0, i, 0))
_dinv_spec = pl.BlockSpec((ROWBLK, HALF), lambda i: (i, 0))
_w_spec = pl.BlockSpec((D, D), lambda i: (0, 0))
_b_spec = pl.BlockSpec((1, D), lambda i: (0, 0))
_degp_spec = pl.BlockSpec((2, ROWBLK, _DEG_W), lambda i: (0, i, 0))

_tca = pl.pallas_call(
    _tca_body,
    grid=(_GRID,),
    in_specs=[_row_spec, _w_spec, _degp_spec],
    out_specs=[_half2_spec, _dinv_spec],
    out_shape=[
        jax.ShapeDtypeStruct((2, N, HALF), jnp.float32),
        jax.ShapeDtypeStruct((N, HALF), jnp.float32),
    ],
)

_tcb = pl.pallas_call(
    _tcb_body,
    grid=(_GRID,),
    in_specs=[_half2_spec, _half2_spec, _dinv_spec, _w_spec, _b_spec],
    out_specs=_half2_spec,
    out_shape=jax.ShapeDtypeStruct((2, N, HALF), jnp.float32),
)

_tcc = pl.pallas_call(
    _tcc_body,
    grid=(_GRID,),
    in_specs=[_half2_spec, _half2_spec, _dinv_spec, _b_spec],
    out_specs=_row_spec,
    out_shape=jax.ShapeDtypeStruct((N, D), jnp.float32),
)


@jax.jit
def kernel(x, edge_index, W1, b1, W2, b2):
    src = edge_index[0].astype(jnp.int32)
    dst = edge_index[1].astype(jnp.int32)
    b1r = b1.reshape(1, D)
    b2r = b2.reshape(1, D)

    degp = _deg_kernel(dst).reshape(2, N, _DEG_W)
    hhat, dinv = _tca(x, W1, degp)
    agg1 = _agg_kernel(hhat.reshape(NCORE * N, HALF), src, dst)
    hhat2 = _tcb(agg1.reshape(2, N, HALF), hhat, dinv, W2, b1r)
    agg2 = _agg_kernel(hhat2.reshape(NCORE * N, HALF), src, dst)
    return _tcc(agg2.reshape(2, N, HALF), hhat2, dinv, b2r)


# speedup vs baseline: 18.5393x; 1.0031x over previous
import functools

import jax
import jax.numpy as jnp
from jax import lax
from jax.experimental import pallas as pl
from jax.experimental.pallas import tpu as pltpu
import jax.experimental.pallas.tpu_sc as plsc

N = 10000
E = 160000
D = 256
HALF = 128
CH = 128
NCHUNK = E // CH
NCORE = 2
NSUB = 16
BLK = 80
NBLK = N // BLK
ROWBLK = 1000

_mesh = plsc.VectorSubcoreMesh(core_axis_name="c", subcore_axis_name="s")


def _zero_fill(ref, rows, cols):
    per_row = cols // 16

    def body(k, _):
        ref[k // per_row, pl.ds((k % per_row) * 16, 16)] = jnp.zeros(
            (16,), jnp.float32)
        return 0

    lax.fori_loop(0, rows * per_row, body, 0)


def _nsplit(total, s):
    return jnp.where(s < (total % NSUB), total // NSUB + 1, total // NSUB)


_DEG_W = HALF
_DNB = 3


@functools.partial(
    pl.kernel,
    out_type=jax.ShapeDtypeStruct((NCORE * N, _DEG_W), jnp.float32),
    mesh=_mesh,
    scratch_types=[
        [pltpu.VMEM((CH,), jnp.int32)] * _DNB,
        pltpu.VMEM((CH, HALF), jnp.float32),
        pltpu.VMEM((BLK, HALF), jnp.float32),
        pltpu.VMEM_SHARED((N, HALF), jnp.float32),
        [pltpu.SemaphoreType.DMA] * _DNB,
        [pltpu.SemaphoreType.DMA] * _DNB,
    ],
)
def _deg_kernel(dst_hbm, out_hbm, dst_vs, ones_v, z_v, acc_sh,
                isems, ssems):
    c = lax.axis_index("c")
    s = lax.axis_index("s")
    _zero_fill(z_v, BLK, HALF)

    def ones_body(k, _):
        ones_v[k // 8, pl.ds((k % 8) * 16, 16)] = jnp.ones((16,), jnp.float32)
        return 0

    lax.fori_loop(0, CH * 8, ones_body, 0)

    def zero_acc(k, _):
        off = pl.multiple_of((k * NSUB + s) * BLK, 8)
        pltpu.sync_copy(z_v, acc_sh.at[pl.ds(off, BLK)])
        return 0

    lax.fori_loop(0, _nsplit(NBLK, s), zero_acc, 0)
    plsc.subcore_barrier()

    half = NCHUNK // NCORE
    n = _nsplit(half, s)

    def fire_idx(item, b):
        off = pl.multiple_of((c * half + item * NSUB + s) * CH, 8)
        pltpu.async_copy(dst_hbm.at[pl.ds(off, CH)], dst_vs[b], isems[b])

    for b in range(_DNB):
        pl.when(b < n)(lambda b=b: fire_idx(jnp.int32(b), b))

    def body(k, _):
        for b in range(_DNB):
            item = _DNB * k + b

            def step(b=b, item=item):
                pltpu.make_async_copy(
                    dst_hbm.at[pl.ds(0, CH)], dst_vs[b], isems[b]).wait()
                pltpu.async_copy(
                    ones_v, acc_sh.at[dst_vs[b]], ssems[b], add=True)

                def refill():
                    pltpu.make_async_copy(
                        ones_v, acc_sh.at[dst_vs[b]], ssems[b]).wait()
                    fire_idx(item + _DNB, b)

                pl.when(item + _DNB < n)(refill)

            pl.when(item < n)(step)
        return 0

    max_n = half // NSUB + 1
    lax.fori_loop(0, (max_n + _DNB - 1) // _DNB, body, 0)
    for b in range(_DNB):
        def drain(b=b, nn=n):
            pltpu.make_async_copy(ones_v, acc_sh.at[dst_vs[b]],
                                  ssems[b]).wait()
        pl.when(jnp.int32(b) < jnp.minimum(n, _DNB))(drain)
    plsc.subcore_barrier()

    def wout(k, _):
        off = pl.multiple_of((k * NSUB + s) * BLK, 8)
        dst_off = pl.multiple_of(c * N + (k * NSUB + s) * BLK, 8)
        pltpu.sync_copy(acc_sh.at[pl.ds(off, BLK)],
                        out_hbm.at[pl.ds(dst_off, BLK)])
        return 0

    lax.fori_loop(0, _nsplit(NBLK, s), wout, 0)


_NROW = 2
_NIDX = 4


@functools.partial(
    pl.kernel,
    out_type=jax.ShapeDtypeStruct((NCORE * N, HALF), jnp.float32),
    mesh=_mesh,
    scratch_types=[
        [pltpu.VMEM((CH,), jnp.int32)] * _NIDX,
        [pltpu.VMEM((CH,), jnp.int32)] * _NIDX,
        [pltpu.VMEM((CH, HALF), jnp.float32)] * _NROW,
        pltpu.VMEM_SHARED((N, HALF), jnp.float32),
        [pltpu.SemaphoreType.DMA] * _NROW,
        [pltpu.SemaphoreType.DMA] * _NROW,
        [pltpu.SemaphoreType.DMA] * _NIDX,
    ],
)
def _agg_kernel(tab_hbm, src_hbm, dst_hbm, out_hbm,
                src_vs, dst_vs, rows_vs, acc_sh, gsems, ssems, isems):
    c = lax.axis_index("c")
    s = lax.axis_index("s")
    _zero_fill(rows_vs[0], BLK, HALF)

    def zero_acc(k, _):
        off = pl.multiple_of((k * NSUB + s) * BLK, 8)
        pltpu.sync_copy(rows_vs[0].at[pl.ds(0, BLK)],
                        acc_sh.at[pl.ds(off, BLK)])
        return 0

    lax.fori_loop(0, _nsplit(NBLK, s), zero_acc, 0)
    plsc.subcore_barrier()

    n = _nsplit(NCHUNK, s)

    def fire_idx(item, i4):
        off = pl.multiple_of((item * NSUB + s) * CH, 8)
        pltpu.async_copy(src_hbm.at[pl.ds(c * E + off, CH)], src_vs[i4],
                         isems[i4])
        pltpu.async_copy(dst_hbm.at[pl.ds(off, CH)], dst_vs[i4], isems[i4])

    def fire_gather(i4, r2):
        pltpu.make_async_copy(
            src_hbm.at[pl.ds(0, CH)], src_vs[i4], isems[i4]).wait()
        pltpu.make_async_copy(
            src_hbm.at[pl.ds(0, CH)], dst_vs[i4], isems[i4]).wait()
        pltpu.async_copy(tab_hbm.at[src_vs[i4]], rows_vs[r2], gsems[r2])

    def wait_gather_fire_scatter(i4, r2):
        pltpu.make_async_copy(
            tab_hbm.at[src_vs[i4]], rows_vs[r2], gsems[r2]).wait()
        pltpu.async_copy(rows_vs[r2], acc_sh.at[dst_vs[i4]], ssems[r2],
                         add=True)

    def wait_scatter(r2, i4):
        pltpu.make_async_copy(
            rows_vs[r2], acc_sh.at[dst_vs[i4]], ssems[r2]).wait()

    fire_idx(jnp.int32(0), 0)
    fire_idx(jnp.int32(1), 1)
    fire_idx(jnp.int32(2), 2)
    fire_gather(0, 0)
    fire_gather(1, 1)
    wait_gather_fire_scatter(0, 0)
    wait_scatter(0, 0)
    fire_idx(jnp.int32(3), 3)
    fire_idx(jnp.int32(4), 0)
    fire_gather(2, 0)
    wait_gather_fire_scatter(1, 1)

    def body(k, _):
        for jj in range(_NIDX):
            base_item = _NIDX * k + jj + 2
            i4 = (jj + 2) % _NIDX
            r2 = jj % _NROW

            def step(item=base_item, i4=i4, r2=r2):
                ni4 = (i4 + 1) % _NIDX
                nr2 = (r2 + 1) % _NROW

                def prep_next():
                    pi4 = (i4 + 3) % _NIDX
                    wait_scatter(nr2, pi4)
                    pl.when(item + 3 < n)(
                        lambda: fire_idx(item + 3, pi4))
                    fire_gather(ni4, nr2)

                pl.when(item + 1 < n)(prep_next)
                wait_gather_fire_scatter(i4, r2)

            pl.when(base_item < n)(step)
        return 0

    max_n = NCHUNK // NSUB + 1
    lax.fori_loop(0, (max_n - 2 + _NIDX - 1) // _NIDX, body, 0)

    for m in range(_NIDX):
        pl.when((n - 2) % _NIDX == m)(
            lambda m=m: wait_scatter(m % _NROW, m))
    for m in range(_NIDX):
        pl.when((n - 1) % _NIDX == m)(
            lambda m=m: wait_scatter(m % _NROW, m))
    plsc.subcore_barrier()

    def wout(k, _):
        off = pl.multiple_of((k * NSUB + s) * BLK, 8)
        dst_off = pl.multiple_of(c * N + (k * NSUB + s) * BLK, 8)
        pltpu.sync_copy(acc_sh.at[pl.ds(off, BLK)],
                        out_hbm.at[pl.ds(dst_off, BLK)])
        return 0

    lax.fori_loop(0, _nsplit(NBLK, s), wout, 0)


def _mm_body(x_ref, w_ref, h_ref):
    h_ref[...] = jnp.dot(x_ref[...], w_ref[...],
                         preferred_element_type=jnp.float32)


def _tca_body(h_ref, degp_ref, hhat_ref, dinv_ref):
    degp = degp_ref[...]
    deg = degp[0, :, 0] + degp[1, :, 0] + 1.0
    dinv = lax.rsqrt(deg)[:, None]
    hh = h_ref[...] * dinv
    hhat_ref[0] = hh[:, :HALF]
    hhat_ref[1] = hh[:, HALF:]
    dinv_ref[...] = jnp.broadcast_to(dinv, (ROWBLK, HALF))


def _tcb_body(agg_ref, hhat_ref, dinv_ref, w_ref, b_ref, hhat2_ref):
    agg = jnp.concatenate([agg_ref[0] + hhat_ref[0],
                           agg_ref[1] + hhat_ref[1]], axis=1)
    dinv = dinv_ref[:, :1]
    u = jnp.maximum(agg * dinv + b_ref[...], 0.0)
    h2 = jnp.dot(u, w_ref[...], preferred_element_type=jnp.float32)
    hh2 = h2 * dinv
    hhat2_ref[0] = hh2[:, :HALF]
    hhat2_ref[1] = hh2[:, HALF:]


def _tcc_body(agg_ref, hhat_ref, dinv_ref, b_ref, out_ref):
    agg = jnp.concatenate([agg_ref[0] + hhat_ref[0],
                           agg_ref[1] + hhat_ref[1]], axis=1)
    dinv = dinv_ref[:, :1]
    out_ref[...] = agg * dinv + b_ref[...]


_GRID = N // ROWBLK

_row_spec = pl.BlockSpec((ROWBLK, D), lambda i: (i, 0))
_half2_spec = pl.BlockSpec((2, ROWBLK, HALF), lambda i: (0, i, 0))
_dinv_spec = pl.BlockSpec((ROWBLK, HALF), lambda i: (i, 0))
_w_spec = pl.BlockSpec((D, D), lambda i: (0, 0))
_b_spec = pl.BlockSpec((1, D), lambda i: (0, 0))
_degp_spec = pl.BlockSpec((2, ROWBLK, _DEG_W), lambda i: (0, i, 0))

_mm = pl.pallas_call(
    _mm_body,
    grid=(_GRID,),
    in_specs=[_row_spec, _w_spec],
    out_specs=_row_spec,
    out_shape=jax.ShapeDtypeStruct((N, D), jnp.float32),
)

_tca = pl.pallas_call(
    _tca_body,
    grid=(_GRID,),
    in_specs=[_row_spec, _degp_spec],
    out_specs=[_half2_spec, _dinv_spec],
    out_shape=[
        jax.ShapeDtypeStruct((2, N, HALF), jnp.float32),
        jax.ShapeDtypeStruct((N, HALF), jnp.float32),
    ],
)

_tcb = pl.pallas_call(
    _tcb_body,
    grid=(_GRID,),
    in_specs=[_half2_spec, _half2_spec, _dinv_spec, _w_spec, _b_spec],
    out_specs=_half2_spec,
    out_shape=jax.ShapeDtypeStruct((2, N, HALF), jnp.float32),
)

_tcc = pl.pallas_call(
    _tcc_body,
    grid=(_GRID,),
    in_specs=[_half2_spec, _half2_spec, _dinv_spec, _b_spec],
    out_specs=_row_spec,
    out_shape=jax.ShapeDtypeStruct((N, D), jnp.float32),
)


@jax.jit
def kernel(x, edge_index, W1, b1, W2, b2):
    src = edge_index[0].astype(jnp.int32)
    dst = edge_index[1].astype(jnp.int32)
    src2 = jnp.concatenate([src, src + N])
    b1r = b1.reshape(1, D)
    b2r = b2.reshape(1, D)

    degp = _deg_kernel(dst).reshape(2, N, _DEG_W)
    h1 = _mm(x, W1)
    hhat, dinv = _tca(h1, degp)
    agg1 = _agg_kernel(hhat.reshape(NCORE * N, HALF), src2, dst)
    hhat2 = _tcb(agg1.reshape(2, N, HALF), hhat, dinv, W2, b1r)
    agg2 = _agg_kernel(hhat2.reshape(NCORE * N, HALF), src2, dst)
    return _tcc(agg2.reshape(2, N, HALF), hhat2, dinv, b2r)
